# baseline (speedup 1.0000x reference)
import functools

import jax
import jax.numpy as jnp
from jax import lax
from jax.experimental import pallas as pl
from jax.experimental.pallas import tpu as pltpu
from jax.experimental.pallas import tpu_sc as plsc

_B = 16384
_D = 64
_NC = 2
_NS = 16
_NW = _NC * _NS
_BPW = _B // _NW
_CH = 128
_NCH = _BPW // _CH
_L = 16


def _dot_kernel(uid_hbm, iid_hbm, ut_hbm, it_hbm, out_hbm,
                iu_v, ii_v, ru_v, ri_v, o_v, sem):
    wid = lax.axis_index("s") * _NC + lax.axis_index("c")
    base = wid * _BPW

    for j in range(_NCH):
        pltpu.sync_copy(uid_hbm.at[pl.ds(base + j * _CH, _CH)], iu_v.at[j])
        pltpu.sync_copy(iid_hbm.at[pl.ds(base + j * _CH, _CH)], ii_v.at[j])

    copies = []
    for j in range(_NCH):
        copies.append(pltpu.async_copy(
            ut_hbm.at[iu_v.at[j]], ru_v.at[pl.ds(j * _CH, _CH)], sem))
        copies.append(pltpu.async_copy(
            it_hbm.at[ii_v.at[j]], ri_v.at[pl.ds(j * _CH, _CH)], sem))
    for c in copies:
        c.wait()

    lanes = lax.iota(jnp.int32, _L)
    masks = {s: (lanes & s) == 0 for s in (8, 4, 2, 1)}
    perms = {s: lanes ^ s for s in (8, 4, 2, 1)}
    bitrev = (((lanes & 1) << 3) | ((lanes & 2) << 1)
              | ((lanes & 4) >> 1) | ((lanes & 8) >> 3))

    def swap(x, s):
        return x.at[perms[s]].get(mode="promise_in_bounds")

    def combine(a, b, s):
        return (jnp.where(masks[s], a, swap(b, s))
                + jnp.where(masks[s], swap(a, s), b))

    def body(g, carry):
        vecs = []
        for j in range(_L):
            r = g * _L + j
            p = ru_v[r, pl.ds(0, _L)] * ri_v[r, pl.ds(0, _L)]
            for c in range(1, _D // _L):
                p = p + ru_v[r, pl.ds(c * _L, _L)] * ri_v[r, pl.ds(c * _L, _L)]
            vecs.append(p)
        for s in (8, 4, 2, 1):
            vecs = [combine(vecs[2 * i], vecs[2 * i + 1], s)
                    for i in range(len(vecs) // 2)]
        o_v[pl.ds(g * _L, _L)] = vecs[0].at[bitrev].get(
            mode="promise_in_bounds")
        return carry

    lax.fori_loop(0, _BPW // _L, body, 0)

    pltpu.sync_copy(o_v, out_hbm.at[pl.ds(base, _BPW)])


@jax.jit
def kernel(user_ids, item_ids, user_table, item_table):
    run = functools.partial(
        pl.kernel,
        mesh=plsc.VectorSubcoreMesh(core_axis_name="c", subcore_axis_name="s"),
        out_type=jax.ShapeDtypeStruct((_B,), jnp.float32),
        scratch_types=[
            pltpu.VMEM((_NCH, _CH), jnp.int32),
            pltpu.VMEM((_NCH, _CH), jnp.int32),
            pltpu.VMEM((_BPW, _D), jnp.float32),
            pltpu.VMEM((_BPW, _D), jnp.float32),
            pltpu.VMEM((_BPW,), jnp.float32),
            pltpu.SemaphoreType.DMA,
        ],
        compiler_params=pltpu.CompilerParams(use_tc_tiling_on_sc=False),
    )(_dot_kernel)
    return run(user_ids, item_ids, user_table, item_table).reshape(_B, 1)
